# baseline (speedup 1.0000x reference)
import functools

import jax
import jax.numpy as jnp
from jax import lax
from jax.experimental import pallas as pl
from jax.experimental.pallas import tpu as pltpu
from jax.experimental.pallas import tpu_sc as plsc

N = 10000
E = 320000
D = 128
NC, NS = 2, 16
NW = NC * NS
RPT = 640
N_PAD = NS * RPT
CH = 64
CPW = 160
G = 16
E_PAD = NW * CPW * CH
SS = 4
SR = CH // SS
TC_R = 1000


def _sc_agg_body(x_hbm, src_hbm, dst_hbm, znd_hbm, agg_hbm,
                 src_v, dst_v, rows_a, rows_b, acc, *sems):
    sem_a = sems[:SS]
    sem_b = sems[SS:]
    cid = lax.axis_index("c")
    sid = lax.axis_index("s")
    wid = cid * NS + sid
    r0 = sid * RPT
    pltpu.sync_copy(znd_hbm, rows_a)

    def zinit(k, carry):
        pltpu.sync_copy(rows_a, acc.at[pl.ds(r0 + k * CH, CH)])
        return carry

    lax.fori_loop(0, RPT // CH, zinit, 0)
    row0 = wid * CPW
    plsc.subcore_barrier()

    def group(gi, carry):
        pltpu.sync_copy(src_hbm.at[pl.ds(row0 + gi * G, G)], src_v)
        pltpu.sync_copy(dst_hbm.at[pl.ds(row0 + gi * G, G)], dst_v)
        def fire(j, buf, sem):
            for k in range(SS):
                pltpu.async_copy(
                    x_hbm.at[src_v.at[j, pl.ds(k * SR, SR)]],
                    buf.at[pl.ds(k * SR, SR)], sem[k])

        def drain(j, buf, sem):
            for k in range(SS):
                pltpu.make_async_copy(
                    x_hbm.at[src_v.at[j, pl.ds(k * SR, SR)]],
                    buf.at[pl.ds(k * SR, SR)], sem[k]).wait()

        fire(0, rows_a, sem_a)

        def pair(i, c2):
            j0 = 2 * i
            fire(j0 + 1, rows_b, sem_b)
            drain(j0, rows_a, sem_a)
            pltpu.sync_copy(rows_a, acc.at[dst_v.at[j0]], add=True)

            @pl.when(i + 1 < G // 2)
            def _():
                fire(j0 + 2, rows_a, sem_a)

            drain(j0 + 1, rows_b, sem_b)
            pltpu.sync_copy(rows_b, acc.at[dst_v.at[j0 + 1]], add=True)
            return c2

        lax.fori_loop(0, G // 2, pair, 0)
        return carry

    lax.fori_loop(0, CPW // G, group, 0)
    plsc.subcore_barrier()

    def out(k, carry):
        r = r0 + k * CH
        pltpu.sync_copy(acc.at[pl.ds(r, CH)], rows_a)
        pltpu.sync_copy(rows_a, agg_hbm.at[cid, pl.ds(r, CH)])
        return carry

    lax.fori_loop(0, RPT // CH, out, 0)


def _make_sc_agg():
    mesh = plsc.VectorSubcoreMesh(core_axis_name="c", subcore_axis_name="s")
    return pl.kernel(
        _sc_agg_body,
        out_type=jax.ShapeDtypeStruct((NC, N_PAD, D), jnp.float32),
        mesh=mesh,
        scratch_types=[
            pltpu.VMEM((G, CH), jnp.int32),
            pltpu.VMEM((G, CH), jnp.int32),
            pltpu.VMEM((CH, D), jnp.float32),
            pltpu.VMEM((CH, D), jnp.float32),
            pltpu.VMEM_SHARED((N_PAD, D), jnp.float32),
        ] + [pltpu.SemaphoreType.DMA] * (2 * SS),
    )


def _sc_deg_body(dst_hbm, znd_hbm, ones_hbm, deg_hbm,
                 dst_v, rows_v, acc, sem):
    cid = lax.axis_index("c")
    sid = lax.axis_index("s")
    wid = cid * NS + sid
    r0 = sid * RPT
    pltpu.sync_copy(znd_hbm, rows_v)

    def zinit(k, carry):
        pltpu.sync_copy(rows_v, acc.at[pl.ds(r0 + k * CH, CH)])
        return carry

    lax.fori_loop(0, RPT // CH, zinit, 0)
    pltpu.sync_copy(ones_hbm, rows_v)
    row0 = wid * CPW
    plsc.subcore_barrier()

    def group(gi, carry):
        pltpu.sync_copy(dst_hbm.at[pl.ds(row0 + gi * G, G)], dst_v)

        def chunk(j, c2):
            pltpu.sync_copy(rows_v, acc.at[dst_v.at[j]], add=True)
            return c2

        lax.fori_loop(0, G, chunk, 0)
        return carry

    lax.fori_loop(0, CPW // G, group, 0)
    plsc.subcore_barrier()

    def out(k, carry):
        r = r0 + k * CH
        pltpu.sync_copy(acc.at[pl.ds(r, CH)], rows_v)
        pltpu.sync_copy(rows_v, deg_hbm.at[cid, pl.ds(r, CH)])
        return carry

    lax.fori_loop(0, RPT // CH, out, 0)


def _make_sc_deg():
    mesh = plsc.VectorSubcoreMesh(core_axis_name="c", subcore_axis_name="s")
    return pl.kernel(
        _sc_deg_body,
        out_type=jax.ShapeDtypeStruct((NC, N_PAD, D), jnp.float32),
        mesh=mesh,
        scratch_types=[
            pltpu.VMEM((G, CH), jnp.int32),
            pltpu.VMEM((CH, D), jnp.float32),
            pltpu.VMEM_SHARED((N_PAD, D), jnp.float32),
            pltpu.SemaphoreType.DMA,
        ],
    )


def _tc_body(relu, parts_ref, x_ref, degp_ref, w_ref, b_ref, o_ref):
    s = parts_ref[0] + parts_ref[1] + x_ref[...]
    d = degp_ref[0, :, 0:1] + degp_ref[1, :, 0:1]
    h = s / (d + 1.0)
    y = jnp.dot(h, w_ref[...], preferred_element_type=jnp.float32) + b_ref[...]
    o_ref[...] = jnp.maximum(y, 0.0) if relu else y


def _tc_layer(x, agg, degp, Wm, bm, relu):
    return pl.pallas_call(
        functools.partial(_tc_body, relu),
        grid=(N // TC_R,),
        in_specs=[
            pl.BlockSpec((NC, TC_R, D), lambda i: (0, i, 0)),
            pl.BlockSpec((TC_R, D), lambda i: (i, 0)),
            pl.BlockSpec((NC, TC_R, D), lambda i: (0, i, 0)),
            pl.BlockSpec((D, D), lambda i: (0, 0)),
            pl.BlockSpec((1, D), lambda i: (0, 0)),
        ],
        out_specs=pl.BlockSpec((TC_R, D), lambda i: (i, 0)),
        out_shape=jax.ShapeDtypeStruct((N, D), jnp.float32),
    )(agg, x, degp, Wm, bm)


def kernel(g, features, W1, b1, W2, b2):
    src, dst = g[0], g[1]
    pad = E_PAD - E
    perm = jnp.argsort(src)
    src = jnp.take(src, perm)
    dst = jnp.take(dst, perm)
    src2d = jnp.concatenate(
        [src, jnp.zeros((pad,), jnp.int32)]).reshape(E_PAD // CH, CH)
    dst2d = jnp.concatenate(
        [dst, jnp.full((pad,), N, jnp.int32)]).reshape(E_PAD // CH, CH)
    zeros_nd = jnp.zeros((CH, D), jnp.float32)
    ones_nd = jnp.ones((CH, D), jnp.float32)

    sc_agg = _make_sc_agg()
    sc_deg = _make_sc_deg()
    agg1 = sc_agg(features, src2d, dst2d, zeros_nd)
    degp = sc_deg(dst2d, zeros_nd, ones_nd)
    h1 = _tc_layer(features, agg1, degp, W1, b1.reshape(1, D), relu=True)
    agg2 = sc_agg(h1, src2d, dst2d, zeros_nd)
    return _tc_layer(h1, agg2, degp, W2, b2.reshape(1, D), relu=False)


# speedup vs baseline: 1.3134x; 1.3134x over previous
import functools

import jax
import jax.numpy as jnp
from jax import lax
from jax.experimental import pallas as pl
from jax.experimental.pallas import tpu as pltpu
from jax.experimental.pallas import tpu_sc as plsc

N = 10000
E = 320000
D = 128
NC, NS = 2, 16
NW = NC * NS
RPT = 640
N_PAD = NS * RPT
CH = 64
CPW = 160
G = 16
E_PAD = NW * CPW * CH
SS = 4
SR = CH // SS
TC_R = 1000


def _sc_agg_body(x_hbm, src_hbm, dst_hbm, znd_hbm, agg_hbm,
                 src_v, dst_v, rows_a, rows_b, acc, *sems):
    sem_a = sems[:SS]
    sem_b = sems[SS:]
    cid = lax.axis_index("c")
    sid = lax.axis_index("s")
    wid = cid * NS + sid
    r0 = sid * RPT
    pltpu.sync_copy(znd_hbm.at[pl.ds(r0, RPT)], acc.at[pl.ds(r0, RPT)])
    row0 = wid * CPW
    plsc.subcore_barrier()

    def group(gi, carry):
        pltpu.sync_copy(src_hbm.at[pl.ds(row0 + gi * G, G)], src_v)
        pltpu.sync_copy(dst_hbm.at[pl.ds(row0 + gi * G, G)], dst_v)
        def fire(j, buf, sem):
            for k in range(SS):
                pltpu.async_copy(
                    x_hbm.at[src_v.at[j, pl.ds(k * SR, SR)]],
                    buf.at[pl.ds(k * SR, SR)], sem[k])

        def drain(j, buf, sem):
            for k in range(SS):
                pltpu.make_async_copy(
                    x_hbm.at[src_v.at[j, pl.ds(k * SR, SR)]],
                    buf.at[pl.ds(k * SR, SR)], sem[k]).wait()

        fire(0, rows_a, sem_a)

        def pair(i, c2):
            j0 = 2 * i
            fire(j0 + 1, rows_b, sem_b)
            drain(j0, rows_a, sem_a)
            pltpu.sync_copy(rows_a, acc.at[dst_v.at[j0]], add=True)

            @pl.when(i + 1 < G // 2)
            def _():
                fire(j0 + 2, rows_a, sem_a)

            drain(j0 + 1, rows_b, sem_b)
            pltpu.sync_copy(rows_b, acc.at[dst_v.at[j0 + 1]], add=True)
            return c2

        lax.fori_loop(0, G // 2, pair, 0)
        return carry

    lax.fori_loop(0, CPW // G, group, 0)
    plsc.subcore_barrier()
    pltpu.sync_copy(acc.at[pl.ds(r0, RPT)], agg_hbm.at[cid, pl.ds(r0, RPT)])


def _make_sc_agg():
    mesh = plsc.VectorSubcoreMesh(core_axis_name="c", subcore_axis_name="s")
    return pl.kernel(
        _sc_agg_body,
        out_type=jax.ShapeDtypeStruct((NC, N_PAD, D), jnp.float32),
        mesh=mesh,
        scratch_types=[
            pltpu.VMEM((G, CH), jnp.int32),
            pltpu.VMEM((G, CH), jnp.int32),
            pltpu.VMEM((CH, D), jnp.float32),
            pltpu.VMEM((CH, D), jnp.float32),
            pltpu.VMEM_SHARED((N_PAD, D), jnp.float32),
        ] + [pltpu.SemaphoreType.DMA] * (2 * SS),
    )


def _sc_deg_body(dst_hbm, znd_hbm, ones_hbm, deg_hbm,
                 dst_v, rows_v, acc, sem):
    cid = lax.axis_index("c")
    sid = lax.axis_index("s")
    wid = cid * NS + sid
    r0 = sid * RPT
    pltpu.sync_copy(znd_hbm.at[pl.ds(r0, RPT)], acc.at[pl.ds(r0, RPT)])
    pltpu.sync_copy(ones_hbm, rows_v)
    row0 = wid * CPW
    plsc.subcore_barrier()

    def group(gi, carry):
        pltpu.sync_copy(dst_hbm.at[pl.ds(row0 + gi * G, G)], dst_v)

        def chunk(j, c2):
            pltpu.sync_copy(rows_v, acc.at[dst_v.at[j]], add=True)
            return c2

        lax.fori_loop(0, G, chunk, 0)
        return carry

    lax.fori_loop(0, CPW // G, group, 0)
    plsc.subcore_barrier()
    pltpu.sync_copy(acc.at[pl.ds(r0, RPT)], deg_hbm.at[cid, pl.ds(r0, RPT)])


def _make_sc_deg():
    mesh = plsc.VectorSubcoreMesh(core_axis_name="c", subcore_axis_name="s")
    return pl.kernel(
        _sc_deg_body,
        out_type=jax.ShapeDtypeStruct((NC, N_PAD, D), jnp.float32),
        mesh=mesh,
        scratch_types=[
            pltpu.VMEM((G, CH), jnp.int32),
            pltpu.VMEM((CH, D), jnp.float32),
            pltpu.VMEM_SHARED((N_PAD, D), jnp.float32),
            pltpu.SemaphoreType.DMA,
        ],
    )


def _tc_body(relu, parts_ref, x_ref, degp_ref, w_ref, b_ref, o_ref):
    s = parts_ref[0] + parts_ref[1] + x_ref[...]
    d = degp_ref[0, :, 0:1] + degp_ref[1, :, 0:1]
    h = s / (d + 1.0)
    y = jnp.dot(h, w_ref[...], preferred_element_type=jnp.float32) + b_ref[...]
    o_ref[...] = jnp.maximum(y, 0.0) if relu else y


def _tc_layer(x, agg, degp, Wm, bm, relu):
    return pl.pallas_call(
        functools.partial(_tc_body, relu),
        grid=(N // TC_R,),
        in_specs=[
            pl.BlockSpec((NC, TC_R, D), lambda i: (0, i, 0)),
            pl.BlockSpec((TC_R, D), lambda i: (i, 0)),
            pl.BlockSpec((NC, TC_R, D), lambda i: (0, i, 0)),
            pl.BlockSpec((D, D), lambda i: (0, 0)),
            pl.BlockSpec((1, D), lambda i: (0, 0)),
        ],
        out_specs=pl.BlockSpec((TC_R, D), lambda i: (i, 0)),
        out_shape=jax.ShapeDtypeStruct((N, D), jnp.float32),
    )(agg, x, degp, Wm, bm)


def kernel(g, features, W1, b1, W2, b2):
    src, dst = g[0], g[1]
    pad = E_PAD - E
    src2d = jnp.concatenate(
        [src, jnp.zeros((pad,), jnp.int32)]).reshape(E_PAD // CH, CH)
    dst2d = jnp.concatenate(
        [dst, jnp.full((pad,), N, jnp.int32)]).reshape(E_PAD // CH, CH)
    zeros_nd = jnp.zeros((N_PAD, D), jnp.float32)
    ones_nd = jnp.ones((CH, D), jnp.float32)

    sc_agg = _make_sc_agg()
    sc_deg = _make_sc_deg()
    agg1 = sc_agg(features, src2d, dst2d, zeros_nd)
    degp = sc_deg(dst2d, zeros_nd, ones_nd)
    h1 = _tc_layer(features, agg1, degp, W1, b1.reshape(1, D), relu=True)
    agg2 = sc_agg(h1, src2d, dst2d, zeros_nd)
    return _tc_layer(h1, agg2, degp, W2, b2.reshape(1, D), relu=False)


# speedup vs baseline: 1.3166x; 1.0025x over previous
import functools

import jax
import jax.numpy as jnp
from jax import lax
from jax.experimental import pallas as pl
from jax.experimental.pallas import tpu as pltpu
from jax.experimental.pallas import tpu_sc as plsc

N = 10000
E = 320000
D = 128
NC, NS = 2, 16
NW = NC * NS
RPT = 640
N_PAD = NS * RPT
CH = 64
CPW = 160
G = 16
E_PAD = NW * CPW * CH
SS = 4
SR = CH // SS
TC_R = 1000


def _sc_agg_body(x_hbm, src_hbm, dst_hbm, znd_hbm, agg_hbm,
                 src_v, dst_v, rows_a, rows_b, acc, *sems):
    sem_a = sems[:SS]
    sem_b = sems[SS:]
    cid = lax.axis_index("c")
    sid = lax.axis_index("s")
    wid = cid * NS + sid
    r0 = sid * RPT
    pltpu.sync_copy(znd_hbm.at[pl.ds(r0, RPT)], acc.at[pl.ds(r0, RPT)])
    row0 = wid * CPW
    plsc.subcore_barrier()

    def group(gi, carry):
        pltpu.sync_copy(src_hbm.at[pl.ds(row0 + gi * G, G)], src_v)
        pltpu.sync_copy(dst_hbm.at[pl.ds(row0 + gi * G, G)], dst_v)
        def fire(j, buf, sem):
            for k in range(SS):
                pltpu.async_copy(
                    x_hbm.at[src_v.at[j, pl.ds(k * SR, SR)]],
                    buf.at[pl.ds(k * SR, SR)], sem[k])

        def drain(j, buf, sem):
            for k in range(SS):
                pltpu.make_async_copy(
                    x_hbm.at[src_v.at[j, pl.ds(k * SR, SR)]],
                    buf.at[pl.ds(k * SR, SR)], sem[k]).wait()

        fire(0, rows_a, sem_a)

        def pair(i, c2):
            j0 = 2 * i
            fire(j0 + 1, rows_b, sem_b)
            drain(j0, rows_a, sem_a)
            pltpu.sync_copy(rows_a, acc.at[dst_v.at[j0]], add=True)

            @pl.when(i + 1 < G // 2)
            def _():
                fire(j0 + 2, rows_a, sem_a)

            drain(j0 + 1, rows_b, sem_b)
            pltpu.sync_copy(rows_b, acc.at[dst_v.at[j0 + 1]], add=True)
            return c2

        lax.fori_loop(0, G // 2, pair, 0)
        return carry

    lax.fori_loop(0, CPW // G, group, 0)
    plsc.subcore_barrier()
    pltpu.sync_copy(acc.at[pl.ds(r0, RPT)], agg_hbm.at[cid, pl.ds(r0, RPT)])


def _make_sc_agg():
    mesh = plsc.VectorSubcoreMesh(core_axis_name="c", subcore_axis_name="s")
    return pl.kernel(
        _sc_agg_body,
        out_type=jax.ShapeDtypeStruct((NC, N_PAD, D), jnp.float32),
        mesh=mesh,
        scratch_types=[
            pltpu.VMEM((G, CH), jnp.int32),
            pltpu.VMEM((G, CH), jnp.int32),
            pltpu.VMEM((CH, D), jnp.float32),
            pltpu.VMEM((CH, D), jnp.float32),
            pltpu.VMEM_SHARED((N_PAD, D), jnp.float32),
        ] + [pltpu.SemaphoreType.DMA] * (2 * SS),
    )


def _sc_deg_body(dst_hbm, znd_hbm, ones_hbm, deg_hbm,
                 dst_v, rows_v, acc, sem):
    cid = lax.axis_index("c")
    sid = lax.axis_index("s")
    wid = cid * NS + sid
    r0 = sid * RPT
    pltpu.sync_copy(znd_hbm.at[pl.ds(r0, RPT)], acc.at[pl.ds(r0, RPT)])
    pltpu.sync_copy(ones_hbm, rows_v)
    row0 = wid * CPW
    plsc.subcore_barrier()

    def group(gi, carry):
        pltpu.sync_copy(dst_hbm.at[pl.ds(row0 + gi * G, G)], dst_v)

        def chunk(j, c2):
            pltpu.async_copy(rows_v, acc.at[dst_v.at[j]], sem, add=True)
            return c2

        lax.fori_loop(0, G, chunk, 0)

        def dr(j, c2):
            pltpu.make_async_copy(rows_v, acc.at[dst_v.at[j]], sem).wait()
            return c2

        lax.fori_loop(0, G, dr, 0)
        return carry

    lax.fori_loop(0, CPW // G, group, 0)
    plsc.subcore_barrier()
    pltpu.sync_copy(acc.at[pl.ds(r0, RPT)], deg_hbm.at[cid, pl.ds(r0, RPT)])


def _make_sc_deg():
    mesh = plsc.VectorSubcoreMesh(core_axis_name="c", subcore_axis_name="s")
    return pl.kernel(
        _sc_deg_body,
        out_type=jax.ShapeDtypeStruct((NC, N_PAD, D), jnp.float32),
        mesh=mesh,
        scratch_types=[
            pltpu.VMEM((G, CH), jnp.int32),
            pltpu.VMEM((CH, D), jnp.float32),
            pltpu.VMEM_SHARED((N_PAD, D), jnp.float32),
            pltpu.SemaphoreType.DMA,
        ],
    )


def _tc_body(relu, parts_ref, x_ref, degp_ref, w_ref, b_ref, o_ref):
    s = parts_ref[0] + parts_ref[1] + x_ref[...]
    d = degp_ref[0, :, 0:1] + degp_ref[1, :, 0:1]
    h = s / (d + 1.0)
    y = jnp.dot(h, w_ref[...], preferred_element_type=jnp.float32) + b_ref[...]
    o_ref[...] = jnp.maximum(y, 0.0) if relu else y


def _tc_layer(x, agg, degp, Wm, bm, relu):
    return pl.pallas_call(
        functools.partial(_tc_body, relu),
        grid=(N // TC_R,),
        in_specs=[
            pl.BlockSpec((NC, TC_R, D), lambda i: (0, i, 0)),
            pl.BlockSpec((TC_R, D), lambda i: (i, 0)),
            pl.BlockSpec((NC, TC_R, D), lambda i: (0, i, 0)),
            pl.BlockSpec((D, D), lambda i: (0, 0)),
            pl.BlockSpec((1, D), lambda i: (0, 0)),
        ],
        out_specs=pl.BlockSpec((TC_R, D), lambda i: (i, 0)),
        out_shape=jax.ShapeDtypeStruct((N, D), jnp.float32),
    )(agg, x, degp, Wm, bm)


def kernel(g, features, W1, b1, W2, b2):
    src, dst = g[0], g[1]
    pad = E_PAD - E
    src2d = jnp.concatenate(
        [src, jnp.zeros((pad,), jnp.int32)]).reshape(E_PAD // CH, CH)
    dst2d = jnp.concatenate(
        [dst, jnp.full((pad,), N, jnp.int32)]).reshape(E_PAD // CH, CH)
    zeros_nd = jnp.zeros((N_PAD, D), jnp.float32)
    ones_nd = jnp.ones((CH, D), jnp.float32)

    sc_agg = _make_sc_agg()
    sc_deg = _make_sc_deg()
    agg1 = sc_agg(features, src2d, dst2d, zeros_nd)
    degp = sc_deg(dst2d, zeros_nd, ones_nd)
    h1 = _tc_layer(features, agg1, degp, W1, b1.reshape(1, D), relu=True)
    agg2 = sc_agg(h1, src2d, dst2d, zeros_nd)
    return _tc_layer(h1, agg2, degp, W2, b2.reshape(1, D), relu=False)
